# baseline (speedup 1.0000x reference)
import jax
import jax.numpy as jnp
from jax import lax
from jax.experimental import pallas as pl

_B, _NQ, _NC = 128, 300, 91
_NF = _NQ * _NC
_NCH = 214
_NFP = _NCH * 128
_KCH = 16
_KP = 304
_KPD = 384
_BIG = 1e9


def _tc_body(prob3_ref, cc_ref, ll_ref, pcnt_ref, tsz_ref,
             scores_ref, labels_ref, tb_ref, b0_ref, b1_ref,
             x0r_ref, x1r_ref, eseq_ref):
    X = prob3_ref[...]
    lif = lax.broadcasted_iota(jnp.int32, (_B, _NCH, 128), 2).astype(
        jnp.float32)
    cif = lax.broadcasted_iota(jnp.int32, (_B, _NCH, 128), 1).astype(
        jnp.float32)
    avs = []
    ais = []
    for _k in range(_KCH):
        mx = jnp.max(X, axis=2, keepdims=True)
        mi = jnp.min(jnp.where(X == mx, lif, _BIG), axis=2,
                     keepdims=True)
        avs.append(mx[:, :, 0])
        ais.append(mi[:, :, 0] + 128.0 * cif[:, :, 0])
        X = jnp.where(lif == mi, -1.0, X)
    pad = jnp.full((_B, 216 - _NCH), -1.0, jnp.float32)
    padi = jnp.full((_B, 216 - _NCH), -2.0, jnp.float32)
    AVq = []
    AIq = []
    for s in range(4):
        c0, c1 = 54 * s, min(54 * (s + 1), _NCH)
        va = [jnp.concatenate([avs[k][:, c0:c1]] +
                              ([pad] if c1 - c0 < 54 else []), axis=1)
              for k in range(_KCH)]
        ia = [jnp.concatenate([ais[k][:, c0:c1]] +
                              ([padi] if c1 - c0 < 54 else []), axis=1)
              for k in range(_KCH)]
        AVq.append(jnp.concatenate(va, axis=1))
        AIq.append(jnp.concatenate(ia, axis=1))
    AV3 = jnp.stack(AVq, axis=1)
    AI3 = jnp.stack(AIq, axis=1)

    col3 = lax.broadcasted_iota(jnp.int32, (1, 1, 128), 2)

    def ext1(k2, carry):
        av, outv, outi = carry
        mx = jnp.max(av, axis=2, keepdims=True)
        mi = jnp.min(jnp.where(av == mx, AI3, _BIG), axis=2,
                     keepdims=True)
        ohk = (col3 == k2).astype(jnp.float32)
        outv = outv + mx * ohk
        outi = outi + mi * ohk
        av = jnp.where(AI3 == mi, -1.0, av)
        return av, outv, outi

    _, tv3, ti3 = lax.fori_loop(
        0, 128, ext1,
        (AV3, jnp.zeros((_B, 4, 128), jnp.float32),
         jnp.zeros((_B, 4, 128), jnp.float32)))

    SV = jnp.concatenate([tv3[:, g, :] for g in range(4)], axis=1)
    SI = jnp.concatenate([ti3[:, g, :] for g in range(4)], axis=1)

    li512 = lax.broadcasted_iota(jnp.int32, (1, 512), 1)
    prow = lax.broadcasted_iota(jnp.int32, (512, 512), 0)
    pcol = lax.broadcasted_iota(jnp.int32, (512, 512), 1)
    kk = 2
    while kk <= 512:
        jj = kk >> 1
        while jj >= 1:
            P = ((prow ^ jj) == pcol).astype(jnp.float32)
            BV = lax.dot_general(SV, P, (((1,), (0,)), ((), ())),
                                 precision=lax.Precision.HIGHEST)
            BI = lax.dot_general(SI, P, (((1,), (0,)), ((), ())),
                                 precision=lax.Precision.HIGHEST)
            less = jnp.logical_or(
                SV > BV, jnp.logical_and(SV == BV, SI < BI))
            want_min = ((li512 & kk) == 0) == ((li512 & jj) == 0)
            keep = want_min == less
            SV = jnp.where(keep, SV, BV)
            SI = jnp.where(keep, SI, BI)
            jj >>= 1
        kk <<= 1
    outv = SV[:, :_KP]
    outi = SI[:, :_KP]

    scores_ref[...] = outv
    fi = outi.astype(jnp.int32)
    tb = fi // _NC
    labels_ref[...] = fi - tb * _NC
    tb_ref[...] = tb

    c = cc_ref[...]
    l = ll_ref[...]
    x0 = c - 0.5 * l
    x1 = c + 0.5 * l
    x0r_ref[...] = x0
    x1r_ref[...] = x1
    s = tsz_ref[...]
    sx0 = jnp.clip(x0, 0.0, 1.0) * s
    sx1 = jnp.clip(x1, 0.0, 1.0) * s
    tbc = jnp.clip(tb, 0, _NQ - 1)
    tio = lax.broadcasted_iota(jnp.int32, (_B, 16, _KPD), 2)
    b0s = []
    b1s = []
    for q0 in range(0, _KP, 16):
        tq = lax.broadcast_in_dim(tbc[:, q0:q0 + 16], (_B, 16, _KPD),
                                  (0, 1))
        oh = (tq == tio).astype(jnp.float32)
        b0s.append(lax.dot_general(
            oh, sx0, (((2,), (1,)), ((0,), (0,))),
            precision=lax.Precision.HIGHEST))
        b1s.append(lax.dot_general(
            oh, sx1, (((2,), (1,)), ((0,), (0,))),
            precision=lax.Precision.HIGHEST))
    b0_ref[...] = jnp.concatenate(b0s, axis=1)
    b1_ref[...] = jnp.concatenate(b1s, axis=1)

    pc = pcnt_ref[...]
    mx2 = jnp.max(pc, axis=1, keepdims=True)
    iot = lax.broadcasted_iota(jnp.int32, pc.shape, 1)
    idx = jnp.min(jnp.where(pc == mx2, iot, jnp.int32(1 << 30)),
                  axis=1, keepdims=True)
    eseq_ref[...] = jnp.maximum(idx, 1)


def _tc_call(prob3, cc, ll, pcnt_p, tsz):
    return pl.pallas_call(
        _tc_body,
        out_shape=[
            jax.ShapeDtypeStruct((_B, _KP), jnp.float32),
            jax.ShapeDtypeStruct((_B, _KP), jnp.int32),
            jax.ShapeDtypeStruct((_B, _KP), jnp.int32),
            jax.ShapeDtypeStruct((_B, _KP), jnp.float32),
            jax.ShapeDtypeStruct((_B, _KP), jnp.float32),
            jax.ShapeDtypeStruct((_B, _KPD), jnp.float32),
            jax.ShapeDtypeStruct((_B, _KPD), jnp.float32),
            jax.ShapeDtypeStruct((_B, 1), jnp.int32),
        ],
    )(prob3, cc, ll, pcnt_p, tsz)


def kernel(pred_logits, pred_boxes, target_sizes, pred_count):
    prob = jax.nn.sigmoid(pred_logits.reshape(_B, _NF))
    prob_p = jnp.pad(prob, ((0, 0), (0, _NFP - _NF)), constant_values=-1.0)
    prob3 = prob_p.reshape(_B, _NCH, 128)
    cc = jnp.pad(pred_boxes[:, :, 0], ((0, 0), (0, _KPD - _NQ)))
    ll = jnp.pad(pred_boxes[:, :, 1], ((0, 0), (0, _KPD - _NQ)))
    pcnt_p = jnp.pad(pred_count, ((0, 0), (0, _KPD - _NQ - 1)),
                     constant_values=-jnp.inf)
    tsz = target_sizes[:, None]

    (scores_p, labels_p, tb_p, b0, b1, x0r, x1r, eseq) = _tc_call(
        prob3, cc, ll, pcnt_p, tsz)

    scores = scores_p[:, :_NQ]
    labels = labels_p[:, :_NQ]
    topk_boxes = tb_p[:, :_NQ]
    boxes = jnp.stack([b0[:, :_NQ], b1[:, :_NQ]], axis=-1)
    raw_boxes = jnp.stack([x0r[:, :_NQ], x1r[:, :_NQ]], axis=-1)
    eseq_lens = eseq[:, 0]
    return scores, labels, boxes, raw_boxes, topk_boxes, eseq_lens


# speedup vs baseline: 1.0495x; 1.0495x over previous
import jax
import jax.numpy as jnp
from jax import lax
from jax.experimental import pallas as pl

_B, _NQ, _NC = 128, 300, 91
_NF = _NQ * _NC
_NCH = 214
_NFP = _NCH * 128
_KCH = 16
_KP = 304
_KPD = 384
_BIG = 1e9


def _tc_body(prob3_ref, cc_ref, ll_ref, pcnt_ref, tsz_ref,
             scores_ref, labels_ref, tb_ref, b0_ref, b1_ref,
             x0r_ref, x1r_ref, eseq_ref):
    X = prob3_ref[...]
    lif = lax.broadcasted_iota(jnp.int32, (_B, _NCH, 128), 2).astype(
        jnp.float32)
    cif = lax.broadcasted_iota(jnp.int32, (_B, _NCH, 128), 1).astype(
        jnp.float32)
    avs = []
    ais = []
    for _k in range(_KCH):
        mx = jnp.max(X, axis=2, keepdims=True)
        mi = jnp.min(jnp.where(X == mx, lif, _BIG), axis=2,
                     keepdims=True)
        avs.append(mx[:, :, 0])
        ais.append(mi[:, :, 0] + 128.0 * cif[:, :, 0])
        X = jnp.where(lif == mi, -1.0, X)
    pad = jnp.full((_B, 216 - _NCH), -1.0, jnp.float32)
    padi = jnp.full((_B, 216 - _NCH), -2.0, jnp.float32)
    AV = jnp.concatenate(
        [jnp.concatenate([avs[k], pad], axis=1) for k in range(_KCH)],
        axis=1)
    AI = jnp.concatenate(
        [jnp.concatenate([ais[k], padi], axis=1) for k in range(_KCH)],
        axis=1)

    col = lax.broadcasted_iota(jnp.int32, (1, _KP), 1)

    def ext(k2, carry):
        av, outv, outi = carry
        mx = jnp.max(av, axis=1, keepdims=True)
        mi = jnp.min(jnp.where(av == mx, AI, _BIG), axis=1,
                     keepdims=True)
        ohk = (col == k2).astype(jnp.float32)
        outv = outv + mx * ohk
        outi = outi + mi * ohk
        av = jnp.where(AI == mi, -1.0, av)
        return av, outv, outi

    _, outv, outi = lax.fori_loop(
        0, _NQ, ext,
        (AV, jnp.zeros((_B, _KP), jnp.float32),
         jnp.zeros((_B, _KP), jnp.float32)))

    scores_ref[...] = outv
    fi = outi.astype(jnp.int32)
    tb = fi // _NC
    labels_ref[...] = fi - tb * _NC
    tb_ref[...] = tb

    c = cc_ref[...]
    l = ll_ref[...]
    x0 = c - 0.5 * l
    x1 = c + 0.5 * l
    x0r_ref[...] = x0
    x1r_ref[...] = x1
    s = tsz_ref[...]
    sx0 = jnp.clip(x0, 0.0, 1.0) * s
    sx1 = jnp.clip(x1, 0.0, 1.0) * s
    tbc = jnp.clip(tb, 0, _NQ - 1)
    tio = lax.broadcasted_iota(jnp.int32, (_B, 16, _KPD), 2)
    b0s = []
    b1s = []
    for q0 in range(0, _KP, 16):
        tq = lax.broadcast_in_dim(tbc[:, q0:q0 + 16], (_B, 16, _KPD),
                                  (0, 1))
        oh = (tq == tio).astype(jnp.float32)
        b0s.append(lax.dot_general(
            oh, sx0, (((2,), (1,)), ((0,), (0,))),
            precision=lax.Precision.HIGHEST))
        b1s.append(lax.dot_general(
            oh, sx1, (((2,), (1,)), ((0,), (0,))),
            precision=lax.Precision.HIGHEST))
    b0_ref[...] = jnp.concatenate(b0s, axis=1)
    b1_ref[...] = jnp.concatenate(b1s, axis=1)

    pc = pcnt_ref[...]
    mx2 = jnp.max(pc, axis=1, keepdims=True)
    iot = lax.broadcasted_iota(jnp.int32, pc.shape, 1)
    idx = jnp.min(jnp.where(pc == mx2, iot, jnp.int32(1 << 30)),
                  axis=1, keepdims=True)
    eseq_ref[...] = jnp.maximum(idx, 1)


def _tc_call(prob3, cc, ll, pcnt_p, tsz):
    return pl.pallas_call(
        _tc_body,
        out_shape=[
            jax.ShapeDtypeStruct((_B, _KP), jnp.float32),
            jax.ShapeDtypeStruct((_B, _KP), jnp.int32),
            jax.ShapeDtypeStruct((_B, _KP), jnp.int32),
            jax.ShapeDtypeStruct((_B, _KP), jnp.float32),
            jax.ShapeDtypeStruct((_B, _KP), jnp.float32),
            jax.ShapeDtypeStruct((_B, _KPD), jnp.float32),
            jax.ShapeDtypeStruct((_B, _KPD), jnp.float32),
            jax.ShapeDtypeStruct((_B, 1), jnp.int32),
        ],
    )(prob3, cc, ll, pcnt_p, tsz)


def kernel(pred_logits, pred_boxes, target_sizes, pred_count):
    prob = jax.nn.sigmoid(pred_logits.reshape(_B, _NF))
    prob_p = jnp.pad(prob, ((0, 0), (0, _NFP - _NF)), constant_values=-1.0)
    prob3 = prob_p.reshape(_B, _NCH, 128)
    cc = jnp.pad(pred_boxes[:, :, 0], ((0, 0), (0, _KPD - _NQ)))
    ll = jnp.pad(pred_boxes[:, :, 1], ((0, 0), (0, _KPD - _NQ)))
    pcnt_p = jnp.pad(pred_count, ((0, 0), (0, _KPD - _NQ - 1)),
                     constant_values=-jnp.inf)
    tsz = target_sizes[:, None]

    (scores_p, labels_p, tb_p, b0, b1, x0r, x1r, eseq) = _tc_call(
        prob3, cc, ll, pcnt_p, tsz)

    scores = scores_p[:, :_NQ]
    labels = labels_p[:, :_NQ]
    topk_boxes = tb_p[:, :_NQ]
    boxes = jnp.stack([b0[:, :_NQ], b1[:, :_NQ]], axis=-1)
    raw_boxes = jnp.stack([x0r[:, :_NQ], x1r[:, :_NQ]], axis=-1)
    eseq_lens = eseq[:, 0]
    return scores, labels, boxes, raw_boxes, topk_boxes, eseq_lens
